# baseline (speedup 1.0000x reference)
import functools

import jax
import jax.numpy as jnp
from jax import lax
from jax.experimental import pallas as pl
from jax.experimental.pallas import tpu as pltpu
from jax.experimental.pallas import tpu_sc as plsc

VOCAB, WIDTH, CTX, BATCH = 49408, 512, 77, 1024
LANES = 16
NUM_CORES = 2
NUM_SUBCORES = 16


CTX_PAD = 80


def kernel(input_ts, tokenembd, pstnembd):
    nw = NUM_CORES * NUM_SUBCORES
    rows_per_w = BATCH // nw

    mesh = plsc.VectorSubcoreMesh(core_axis_name="c", subcore_axis_name="s")

    @functools.partial(
        pl.kernel,
        out_type=jax.ShapeDtypeStruct((BATCH, CTX, WIDTH), jnp.float32),
        mesh=mesh,
        scratch_types=[
            pltpu.VMEM((CTX_PAD,), jnp.int32),
            pltpu.VMEM((CTX, WIDTH), jnp.float32),
            pltpu.VMEM((CTX_PAD, WIDTH), jnp.float32),
            pltpu.SemaphoreType.DMA,
        ],
        compiler_params=pltpu.CompilerParams(use_tc_tiling_on_sc=False),
    )
    def launch(idx_hbm, tok_hbm, pos_hbm, out_hbm, idx_row, pos_v, buf, sem):
        wid = lax.axis_index("s") * NUM_CORES + lax.axis_index("c")
        base = wid * rows_per_w
        pltpu.sync_copy(pos_hbm, pos_v)

        def per_batch_row(b, carry):
            pltpu.sync_copy(idx_hbm.at[base + b], idx_row)
            pltpu.async_copy(tok_hbm.at[idx_row], buf, sem).wait()

            def add_row(r, c2):
                for g in range(WIDTH // LANES):
                    sl = pl.ds(g * LANES, LANES)
                    buf[r, sl] = buf[r, sl] + pos_v[r, sl]
                return c2

            lax.fori_loop(0, CTX, add_row, 0)
            pltpu.sync_copy(buf.at[pl.ds(0, CTX)], out_hbm.at[base + b])
            return carry

        lax.fori_loop(0, rows_per_w, per_batch_row, 0)

    idx_pad = jnp.pad(input_ts.astype(jnp.int32), ((0, 0), (0, CTX_PAD - CTX)))
    return launch(idx_pad, tokenembd, pstnembd)
